# baseline (speedup 1.0000x reference)
import jax
import jax.numpy as jnp
from jax.experimental import pallas as pl

N = 2048
D = 2048
H = 16384
K = 64

_BH_ENC = 512
_BN_SEL = 128
_BN_DEC = 1024
_BH_DEC = 1024


def _encode_kernel(x_ref, w_ref, pb_ref, lb_ref, out_ref):
    xc = x_ref[...] - pb_ref[...]
    out_ref[...] = jax.lax.dot_general(
        xc, w_ref[...], (((1,), (1,)), ((), ())),
        preferred_element_type=jnp.float32) + lb_ref[...]


def _select_kernel(pa_ref, feat_ref):
    pa = pa_ref[...]
    b = jax.lax.bitcast_convert_type(pa, jnp.int32)
    keys = jnp.where(b >= 0, b, b ^ jnp.int32(0x7FFFFFFF))
    lo = jnp.min(keys, axis=1, keepdims=True)
    hi = jnp.max(keys, axis=1, keepdims=True) + 1

    def body(_, c):
        lo, hi = c
        mid = (lo >> 1) + (hi >> 1) + (lo & hi & 1)
        cnt = jnp.sum((keys >= mid).astype(jnp.int32), axis=1, keepdims=True)
        ok = cnt >= K
        return jnp.where(ok, mid, lo), jnp.where(ok, hi, mid)

    lo, _ = jax.lax.fori_loop(0, 32, body, (lo, hi))
    mask = keys >= lo
    feat_ref[...] = jnp.where(mask, jnp.maximum(pa, 0.0), 0.0)


def _decode_kernel(feat_ref, wh_ref, pb_ref, out_ref):
    j = pl.program_id(1)
    acc = jax.lax.dot_general(
        feat_ref[...].astype(jnp.bfloat16), wh_ref[...],
        (((1,), (0,)), ((), ())), preferred_element_type=jnp.float32)

    @pl.when(j == 0)
    def _():
        out_ref[...] = acc + pb_ref[...]

    @pl.when(j != 0)
    def _():
        out_ref[...] += acc


def kernel(x, W, pre_bias, latent_bias):
    w_hi = W.astype(jnp.bfloat16)
    pb = pre_bias.reshape(1, D)
    lb = latent_bias.reshape(1, H)

    pre_act = pl.pallas_call(
        _encode_kernel,
        grid=(H // _BH_ENC,),
        in_specs=[
            pl.BlockSpec((N, D), lambda j: (0, 0)),
            pl.BlockSpec((_BH_ENC, D), lambda j: (j, 0)),
            pl.BlockSpec((1, D), lambda j: (0, 0)),
            pl.BlockSpec((1, _BH_ENC), lambda j: (0, j)),
        ],
        out_specs=pl.BlockSpec((N, _BH_ENC), lambda j: (0, j)),
        out_shape=jax.ShapeDtypeStruct((N, H), jnp.float32),
    )(x, W, pb, lb)

    features = pl.pallas_call(
        _select_kernel,
        grid=(N // _BN_SEL,),
        in_specs=[pl.BlockSpec((_BN_SEL, H), lambda i: (i, 0))],
        out_specs=pl.BlockSpec((_BN_SEL, H), lambda i: (i, 0)),
        out_shape=jax.ShapeDtypeStruct((N, H), jnp.float32),
    )(pre_act)

    x_hat = pl.pallas_call(
        _decode_kernel,
        grid=(N // _BN_DEC, H // _BH_DEC),
        in_specs=[
            pl.BlockSpec((_BN_DEC, _BH_DEC), lambda i, j: (i, j)),
            pl.BlockSpec((_BH_DEC, D), lambda i, j: (j, 0)),
            pl.BlockSpec((1, D), lambda i, j: (0, 0)),
        ],
        out_specs=pl.BlockSpec((_BN_DEC, D), lambda i, j: (i, 0)),
        out_shape=jax.ShapeDtypeStruct((N, D), jnp.float32),
    )(features, w_hi, pb)

    return (x_hat, features)


# speedup vs baseline: 1.1404x; 1.1404x over previous
import jax
import jax.numpy as jnp
from jax.experimental import pallas as pl
from jax.experimental.pallas import tpu as pltpu

N = 2048
D = 2048
H = 16384
K = 64

_BH_ENC = 512
_BN_SEL = 128
_BN_DEC = 1024
_BH_DEC = 1024


def _encode_kernel(x_ref, w_ref, pb_ref, lb_ref, out_ref):
    xc = x_ref[...] - pb_ref[...]
    out_ref[...] = jax.lax.dot_general(
        xc, w_ref[...], (((1,), (1,)), ((), ())),
        preferred_element_type=jnp.float32) + lb_ref[...]


def _select_kernel(pa_ref, feat_ref, lo_ref, hi_ref, done_ref):
    pa = pa_ref[...]
    b = jax.lax.bitcast_convert_type(pa, jnp.int32)
    keys = jnp.where(b >= 0, b, b ^ jnp.int32(0x7FFFFFFF))
    lo0 = jnp.min(keys, axis=1, keepdims=True)
    hi0 = jnp.max(keys, axis=1, keepdims=True) + 1
    done0 = (lo0 + 1 >= hi0).astype(jnp.int32)
    lo_ref[...] = lo0
    hi_ref[...] = hi0
    done_ref[...] = done0

    def cond(n_active):
        return n_active > 0

    def body(_):
        lo = lo_ref[...]
        hi = hi_ref[...]
        done = done_ref[...] > 0
        mid = (lo >> 1) + (hi >> 1) + (lo & hi & 1)
        cnt = jnp.sum((keys >= mid).astype(jnp.int32), axis=1, keepdims=True)
        ok = cnt >= K
        nlo = jnp.where(done, lo, jnp.where(ok, mid, lo))
        nhi = jnp.where(done, hi, jnp.where(ok, hi, mid))
        ndone = done | (~done & (cnt == K)) | (nlo + 1 >= nhi)
        lo_ref[...] = nlo
        hi_ref[...] = nhi
        done_ref[...] = ndone.astype(jnp.int32)
        return jnp.sum(1 - ndone.astype(jnp.int32))

    jax.lax.while_loop(cond, body, jnp.sum(1 - done0))
    mask = keys >= lo_ref[...]
    feat_ref[...] = jnp.where(mask, jnp.maximum(pa, 0.0), 0.0)


def _decode_kernel(feat_ref, w_ref, pb_ref, out_ref):
    j = pl.program_id(1)
    acc = jax.lax.dot_general(
        feat_ref[...], w_ref[...],
        (((1,), (0,)), ((), ())), preferred_element_type=jnp.float32)

    @pl.when(j == 0)
    def _():
        out_ref[...] = acc + pb_ref[...]

    @pl.when(j != 0)
    def _():
        out_ref[...] += acc


def kernel(x, W, pre_bias, latent_bias):
    pb = pre_bias.reshape(1, D)
    lb = latent_bias.reshape(1, H)

    pre_act = pl.pallas_call(
        _encode_kernel,
        grid=(H // _BH_ENC,),
        in_specs=[
            pl.BlockSpec((N, D), lambda j: (0, 0)),
            pl.BlockSpec((_BH_ENC, D), lambda j: (j, 0)),
            pl.BlockSpec((1, D), lambda j: (0, 0)),
            pl.BlockSpec((1, _BH_ENC), lambda j: (0, j)),
        ],
        out_specs=pl.BlockSpec((N, _BH_ENC), lambda j: (0, j)),
        out_shape=jax.ShapeDtypeStruct((N, H), jnp.float32),
    )(x, W, pb, lb)

    features = pl.pallas_call(
        _select_kernel,
        grid=(N // _BN_SEL,),
        in_specs=[pl.BlockSpec((_BN_SEL, H), lambda i: (i, 0))],
        out_specs=pl.BlockSpec((_BN_SEL, H), lambda i: (i, 0)),
        out_shape=jax.ShapeDtypeStruct((N, H), jnp.float32),
        scratch_shapes=[
            pltpu.VMEM((_BN_SEL, 1), jnp.int32),
            pltpu.VMEM((_BN_SEL, 1), jnp.int32),
            pltpu.VMEM((_BN_SEL, 1), jnp.int32),
        ],
    )(pre_act)

    x_hat = pl.pallas_call(
        _decode_kernel,
        grid=(N // _BN_DEC, H // _BH_DEC),
        in_specs=[
            pl.BlockSpec((_BN_DEC, _BH_DEC), lambda i, j: (i, j)),
            pl.BlockSpec((_BH_DEC, D), lambda i, j: (j, 0)),
            pl.BlockSpec((1, D), lambda i, j: (0, 0)),
        ],
        out_specs=pl.BlockSpec((_BN_DEC, D), lambda i, j: (i, 0)),
        out_shape=jax.ShapeDtypeStruct((N, D), jnp.float32),
    )(features, W, pb)

    return (x_hat, features)
